# baseline (speedup 1.0000x reference)
import functools

import jax
import jax.numpy as jnp
from jax import lax
from jax.experimental import pallas as pl
from jax.experimental.pallas import tpu as pltpu
from jax.experimental.pallas import tpu_sc as plsc

N = 10000
NP_ = 10240
E = 320000
D = 128

NC = 2
NS = 16
NW = NC * NS
E_PER_TILE = E // NW
K = 128
CHUNKS = 80
HALF = CHUNKS // 2
ROWS_PER_TILE = NP_ // NS


def _sc_segment_sum(x, src3d, dst3d, z128, ones, with_deg):
    out_types = [jax.ShapeDtypeStruct((NC, NP_, D), jnp.float32)]
    if with_deg:
        out_types.append(jax.ShapeDtypeStruct((NC, NP_, D), jnp.float32))

    scratch = [
        pltpu.VMEM((HALF, K), jnp.int32),
        pltpu.VMEM((HALF, K), jnp.int32),
        pltpu.VMEM((K, D), jnp.float32),
        pltpu.VMEM((K, D), jnp.float32),
        pltpu.VMEM_SHARED((NP_, D), jnp.float32),
        pltpu.SemaphoreType.DMA,
        pltpu.SemaphoreType.DMA,
    ]

    def body(x_hbm, src_hbm, dst_hbm, z128_hbm, ones_hbm, *refs):
        if with_deg:
            parts_hbm, degp_hbm = refs[0], refs[1]
            rest = refs[2:]
        else:
            parts_hbm = refs[0]
            rest = refs[1:]
        src_v, dst_v, rows0, rows1, acc_sh, sg0, sg1 = rest
        rows = (rows0, rows1)
        sg = (sg0, sg1)

        c = lax.axis_index("c")
        s = lax.axis_index("s")
        wid = c * NS + s
        rbase = s * ROWS_PER_TILE

        def zero_acc():
            @pl.loop(0, ROWS_PER_TILE // K)
            def _(j):
                rb = rbase + j * K
                pltpu.sync_copy(z128_hbm.at[pl.ds(rb, K)], rows0)
                pltpu.sync_copy(rows0, acc_sh.at[pl.ds(rb, K)])

        def copy_acc_out(dst_ref):
            @pl.loop(0, ROWS_PER_TILE // K)
            def _(j):
                rb = rbase + j * K
                pltpu.sync_copy(acc_sh.at[pl.ds(rb, K)], rows0)
                pltpu.sync_copy(rows0, dst_ref.at[c].at[pl.ds(rb, K)])

        def load_half(h):
            pltpu.sync_copy(src_hbm.at[wid].at[pl.ds(h * HALF, HALF)], src_v)
            pltpu.sync_copy(dst_hbm.at[wid].at[pl.ds(h * HALF, HALF)], dst_v)

        def issue_gather(i, b):
            pltpu.async_copy(x_hbm.at[src_v.at[i]], rows[b], sg[b])

        def wait_gather(b):
            pltpu.make_async_copy(z128_hbm.at[pl.ds(0, K)], rows[b],
                                  sg[b]).wait()

        def scatter(i, b):
            pltpu.sync_copy(rows[b], acc_sh.at[dst_v.at[i]], add=True)

        zero_acc()
        plsc.subcore_barrier()

        for h in range(2):
            load_half(h)
            issue_gather(0, 0)
            issue_gather(1, 1)

            @pl.loop(0, (HALF - 2) // 2)
            def _(j):
                i = 2 * j
                wait_gather(0)
                scatter(i, 0)
                issue_gather(i + 2, 0)
                wait_gather(1)
                scatter(i + 1, 1)
                issue_gather(i + 3, 1)

            wait_gather(0)
            scatter(HALF - 2, 0)
            wait_gather(1)
            scatter(HALF - 1, 1)

        plsc.subcore_barrier()
        copy_acc_out(parts_hbm)

        if with_deg:
            plsc.subcore_barrier()
            zero_acc()
            pltpu.sync_copy(ones_hbm, rows1)
            plsc.subcore_barrier()

            for h in range(2):
                pltpu.sync_copy(dst_hbm.at[wid].at[pl.ds(h * HALF, HALF)],
                                dst_v)

                @pl.loop(0, HALF // 5)
                def _(g):
                    handles = [
                        pltpu.async_copy(rows1,
                                         acc_sh.at[dst_v.at[5 * g + u]],
                                         sg0, add=True)
                        for u in range(5)
                    ]
                    for hh in handles:
                        hh.wait()

            plsc.subcore_barrier()
            copy_acc_out(degp_hbm)

    mesh = plsc.VectorSubcoreMesh(core_axis_name="c", subcore_axis_name="s")
    kfn = pl.kernel(body, out_type=out_types, mesh=mesh,
                    scratch_types=scratch)
    return kfn(x, src3d, dst3d, z128, ones)


def _tc_body(parts_ref, degp_ref, x_ref, wl_ref, bl_ref, wr_ref, o_ref, *,
             relu):
    p = parts_ref[0] + parts_ref[1]
    dg = degp_ref[0, :, :1] + degp_ref[1, :, :1]
    inv = 1.0 / jnp.maximum(dg, 1.0)
    agg = p * inv
    acc = lax.dot_general(agg, wl_ref[...], (((1,), (1,)), ((), ())),
                          preferred_element_type=jnp.float32)
    acc = acc + bl_ref[...]
    acc = acc + lax.dot_general(x_ref[...], wr_ref[...],
                                (((1,), (1,)), ((), ())),
                                preferred_element_type=jnp.float32)
    o_ref[...] = jnp.maximum(acc, 0.0) if relu else acc


def _tc_combine(parts, degp, x, Wl, bl, Wr, relu):
    R = 2000
    grid = (N // R,)
    return pl.pallas_call(
        functools.partial(_tc_body, relu=relu),
        grid=grid,
        in_specs=[
            pl.BlockSpec((NC, R, D), lambda i: (0, i, 0)),
            pl.BlockSpec((NC, R, D), lambda i: (0, i, 0)),
            pl.BlockSpec((R, D), lambda i: (i, 0)),
            pl.BlockSpec((D, D), lambda i: (0, 0)),
            pl.BlockSpec((1, D), lambda i: (0, 0)),
            pl.BlockSpec((D, D), lambda i: (0, 0)),
        ],
        out_specs=pl.BlockSpec((R, D), lambda i: (i, 0)),
        out_shape=jax.ShapeDtypeStruct((N, D), jnp.float32),
    )(parts, degp, x, Wl, bl.reshape(1, D), Wr)


def kernel(x, edge_index, W1l, b1l, W1r, W2l, b2l, W2r):
    npad = CHUNKS * K - E_PER_TILE
    src2 = edge_index[0].astype(jnp.int32).reshape(NW, E_PER_TILE)
    dst2 = edge_index[1].astype(jnp.int32).reshape(NW, E_PER_TILE)
    src3d = jnp.pad(src2, ((0, 0), (0, npad))).reshape(NW, CHUNKS, K)
    dst3d = jnp.pad(dst2, ((0, 0), (0, npad)),
                    constant_values=NP_ - 1).reshape(NW, CHUNKS, K)
    z128 = jnp.zeros((NP_, D), jnp.float32)
    ones = jnp.ones((K, D), jnp.float32)

    parts1, degp = _sc_segment_sum(x, src3d, dst3d, z128, ones, True)
    h = _tc_combine(parts1, degp, x, W1l, b1l, W1r, relu=True)
    (parts2,) = _sc_segment_sum(h, src3d, dst3d, z128, ones, False)
    out = _tc_combine(parts2, degp, h, W2l, b2l, W2r, relu=False)
    return out


# speedup vs baseline: 1.6689x; 1.6689x over previous
import functools

import jax
import jax.numpy as jnp
from jax import lax
from jax.experimental import pallas as pl
from jax.experimental.pallas import tpu as pltpu
from jax.experimental.pallas import tpu_sc as plsc

N = 10000
NP_ = 10240
E = 320000
D = 128

NC = 2
NS = 16
NW = NC * NS
E_PER_TILE = E // NW
K = 80
CHUNKS = E_PER_TILE // K
ROWS_PER_TILE = NP_ // NS


def _sc_segment_sum(x, src2d, dst3d, z128, ones, with_deg):
    out_types = [jax.ShapeDtypeStruct((NC, NP_, D), jnp.float32)]
    if with_deg:
        out_types.append(jax.ShapeDtypeStruct((NC, NP_, D), jnp.float32))

    scratch = [
        pltpu.VMEM((E_PER_TILE,), jnp.int32),
        pltpu.VMEM((CHUNKS, K), jnp.int32),
        pltpu.VMEM((K, D), jnp.float32),
        pltpu.VMEM_SHARED((NP_, D), jnp.float32),
        pltpu.SemaphoreType.DMA,
    ]

    def body(x_hbm, src_hbm, dst_hbm, z128_hbm, ones_hbm, *refs):
        if with_deg:
            parts_hbm, degp_hbm = refs[0], refs[1]
            rest = refs[2:]
        else:
            parts_hbm = refs[0]
            rest = refs[1:]
        src_v, dst_v, rows_v, acc_sh, sem = rest

        c = lax.axis_index("c")
        s = lax.axis_index("s")
        wid = c * NS + s
        rbase = s * ROWS_PER_TILE

        def zero_acc():
            @pl.loop(0, ROWS_PER_TILE // K)
            def _(j):
                rb = rbase + j * K
                pltpu.sync_copy(z128_hbm.at[pl.ds(rb, K)], rows_v)
                pltpu.sync_copy(rows_v, acc_sh.at[pl.ds(rb, K)])

        def copy_acc_out(dst_ref):
            @pl.loop(0, ROWS_PER_TILE // K)
            def _(j):
                rb = rbase + j * K
                pltpu.sync_copy(acc_sh.at[pl.ds(rb, K)], rows_v)
                pltpu.sync_copy(rows_v, dst_ref.at[c].at[pl.ds(rb, K)])

        pltpu.sync_copy(src_hbm.at[wid], src_v)
        pltpu.sync_copy(dst_hbm.at[wid], dst_v)

        zero_acc()
        plsc.subcore_barrier()

        @pl.loop(0, CHUNKS)
        def _(i):
            pltpu.async_copy(x_hbm.at[src_v.at[pl.ds(i * K, K)]],
                             rows_v, sem).wait()
            pltpu.sync_copy(rows_v, acc_sh.at[dst_v.at[i]], add=True)

        plsc.subcore_barrier()
        copy_acc_out(parts_hbm)

        if with_deg:
            plsc.subcore_barrier()
            zero_acc()
            pltpu.sync_copy(ones_hbm, rows_v)
            plsc.subcore_barrier()

            @pl.loop(0, CHUNKS)
            def _(i):
                pltpu.sync_copy(rows_v, acc_sh.at[dst_v.at[i]], add=True)

            plsc.subcore_barrier()
            copy_acc_out(degp_hbm)

    mesh = plsc.VectorSubcoreMesh(core_axis_name="c", subcore_axis_name="s")
    kfn = pl.kernel(body, out_type=out_types, mesh=mesh,
                    scratch_types=scratch)
    return kfn(x, src2d, dst3d, z128, ones)


def _tc_body(parts_ref, degp_ref, x_ref, wl_ref, bl_ref, wr_ref, o_ref, *,
             relu):
    p = parts_ref[0] + parts_ref[1]
    dg = degp_ref[0, :, :1] + degp_ref[1, :, :1]
    inv = 1.0 / jnp.maximum(dg, 1.0)
    agg = p * inv
    acc = lax.dot_general(agg, wl_ref[...], (((1,), (1,)), ((), ())),
                          preferred_element_type=jnp.float32)
    acc = acc + bl_ref[...]
    acc = acc + lax.dot_general(x_ref[...], wr_ref[...],
                                (((1,), (1,)), ((), ())),
                                preferred_element_type=jnp.float32)
    o_ref[...] = jnp.maximum(acc, 0.0) if relu else acc


def _tc_combine(parts, degp, x, Wl, bl, Wr, relu):
    R = 2000
    grid = (N // R,)
    return pl.pallas_call(
        functools.partial(_tc_body, relu=relu),
        grid=grid,
        in_specs=[
            pl.BlockSpec((NC, R, D), lambda i: (0, i, 0)),
            pl.BlockSpec((NC, R, D), lambda i: (0, i, 0)),
            pl.BlockSpec((R, D), lambda i: (i, 0)),
            pl.BlockSpec((D, D), lambda i: (0, 0)),
            pl.BlockSpec((1, D), lambda i: (0, 0)),
            pl.BlockSpec((D, D), lambda i: (0, 0)),
        ],
        out_specs=pl.BlockSpec((R, D), lambda i: (i, 0)),
        out_shape=jax.ShapeDtypeStruct((N, D), jnp.float32),
    )(parts, degp, x, Wl, bl.reshape(1, D), Wr)


def kernel(x, edge_index, W1l, b1l, W1r, W2l, b2l, W2r):
    src2d = edge_index[0].astype(jnp.int32).reshape(NW, E_PER_TILE)
    dst3d = edge_index[1].astype(jnp.int32).reshape(NW, CHUNKS, K)
    z128 = jnp.zeros((NP_, D), jnp.float32)
    ones = jnp.ones((K, D), jnp.float32)

    parts1, degp = _sc_segment_sum(x, src2d, dst3d, z128, ones, True)
    h = _tc_combine(parts1, degp, x, W1l, b1l, W1r, relu=True)
    (parts2,) = _sc_segment_sum(h, src2d, dst3d, z128, ones, False)
    out = _tc_combine(parts2, degp, h, W2l, b2l, W2r, relu=False)
    return out


# speedup vs baseline: 1.9957x; 1.1958x over previous
import functools

import jax
import jax.numpy as jnp
from jax import lax
from jax.experimental import pallas as pl
from jax.experimental.pallas import tpu as pltpu
from jax.experimental.pallas import tpu_sc as plsc

N = 10000
NP_ = 10240
E = 320000
D = 128

NC = 2
NS = 16
NW = NC * NS
E_PER_TILE = E // NW
K = 80
CHUNKS = E_PER_TILE // K
ROWS_PER_TILE = NP_ // NS


def _sc_segment_sum(x, src2d, dst3d, z128, ones, with_deg):
    out_types = [jax.ShapeDtypeStruct((NC, NP_, D), jnp.float32)]
    if with_deg:
        out_types.append(jax.ShapeDtypeStruct((NC, NP_, D), jnp.float32))

    scratch = [
        pltpu.VMEM((E_PER_TILE,), jnp.int32),
        pltpu.VMEM((CHUNKS, K), jnp.int32),
        pltpu.VMEM((K, D), jnp.float32),
        pltpu.VMEM((K, D), jnp.float32),
        pltpu.VMEM_SHARED((NP_, D), jnp.float32),
        pltpu.SemaphoreType.DMA,
        pltpu.SemaphoreType.DMA,
    ]

    def body(x_hbm, src_hbm, dst_hbm, z128_hbm, ones_hbm, *refs):
        if with_deg:
            parts_hbm, degp_hbm = refs[0], refs[1]
            rest = refs[2:]
        else:
            parts_hbm = refs[0]
            rest = refs[1:]
        src_v, dst_v, rows_v, rows_w, acc_sh, sem0, sem1 = rest

        c = lax.axis_index("c")
        s = lax.axis_index("s")
        wid = c * NS + s
        rbase = s * ROWS_PER_TILE

        def zero_acc():
            @pl.loop(0, ROWS_PER_TILE // K)
            def _(j):
                rb = rbase + j * K
                pltpu.sync_copy(z128_hbm.at[pl.ds(rb, K)], rows_v)
                pltpu.sync_copy(rows_v, acc_sh.at[pl.ds(rb, K)])

        def copy_acc_out(dst_ref):
            @pl.loop(0, ROWS_PER_TILE // K)
            def _(j):
                rb = rbase + j * K
                pltpu.sync_copy(acc_sh.at[pl.ds(rb, K)], rows_v)
                pltpu.sync_copy(rows_v, dst_ref.at[c].at[pl.ds(rb, K)])

        pltpu.sync_copy(src_hbm.at[wid], src_v)
        pltpu.sync_copy(dst_hbm.at[wid], dst_v)

        zero_acc()
        plsc.subcore_barrier()

        @pl.loop(0, CHUNKS // 2)
        def _(j):
            i = 2 * j
            h0 = pltpu.async_copy(x_hbm.at[src_v.at[pl.ds(i * K, K)]],
                                  rows_v, sem0)
            h1 = pltpu.async_copy(x_hbm.at[src_v.at[pl.ds((i + 1) * K, K)]],
                                  rows_w, sem1)
            h0.wait()
            pltpu.sync_copy(rows_v, acc_sh.at[dst_v.at[i]], add=True)
            h1.wait()
            pltpu.sync_copy(rows_w, acc_sh.at[dst_v.at[i + 1]], add=True)

        hl = pltpu.async_copy(x_hbm.at[src_v.at[pl.ds((CHUNKS - 1) * K, K)]],
                              rows_v, sem0)
        hl.wait()
        pltpu.sync_copy(rows_v, acc_sh.at[dst_v.at[CHUNKS - 1]], add=True)

        plsc.subcore_barrier()
        copy_acc_out(parts_hbm)

        if with_deg:
            plsc.subcore_barrier()
            zero_acc()
            pltpu.sync_copy(ones_hbm, rows_v)
            plsc.subcore_barrier()

            @pl.loop(0, CHUNKS // 5)
            def _(g):
                handles = [
                    pltpu.async_copy(rows_v, acc_sh.at[dst_v.at[5 * g + u]],
                                     sem0, add=True)
                    for u in range(5)
                ]
                for hh in handles:
                    hh.wait()

            plsc.subcore_barrier()
            copy_acc_out(degp_hbm)

    mesh = plsc.VectorSubcoreMesh(core_axis_name="c", subcore_axis_name="s")
    kfn = pl.kernel(body, out_type=out_types, mesh=mesh,
                    scratch_types=scratch)
    return kfn(x, src2d, dst3d, z128, ones)


def _tc_body(parts_ref, degp_ref, x_ref, wl_ref, bl_ref, wr_ref, o_ref, *,
             relu):
    p = parts_ref[0] + parts_ref[1]
    dg = degp_ref[0, :, :1] + degp_ref[1, :, :1]
    inv = 1.0 / jnp.maximum(dg, 1.0)
    agg = p * inv
    acc = lax.dot_general(agg, wl_ref[...], (((1,), (1,)), ((), ())),
                          preferred_element_type=jnp.float32)
    acc = acc + bl_ref[...]
    acc = acc + lax.dot_general(x_ref[...], wr_ref[...],
                                (((1,), (1,)), ((), ())),
                                preferred_element_type=jnp.float32)
    o_ref[...] = jnp.maximum(acc, 0.0) if relu else acc


def _tc_combine(parts, degp, x, Wl, bl, Wr, relu):
    R = 2000
    grid = (N // R,)
    return pl.pallas_call(
        functools.partial(_tc_body, relu=relu),
        grid=grid,
        in_specs=[
            pl.BlockSpec((NC, R, D), lambda i: (0, i, 0)),
            pl.BlockSpec((NC, R, D), lambda i: (0, i, 0)),
            pl.BlockSpec((R, D), lambda i: (i, 0)),
            pl.BlockSpec((D, D), lambda i: (0, 0)),
            pl.BlockSpec((1, D), lambda i: (0, 0)),
            pl.BlockSpec((D, D), lambda i: (0, 0)),
        ],
        out_specs=pl.BlockSpec((R, D), lambda i: (i, 0)),
        out_shape=jax.ShapeDtypeStruct((N, D), jnp.float32),
    )(parts, degp, x, Wl, bl.reshape(1, D), Wr)


def kernel(x, edge_index, W1l, b1l, W1r, W2l, b2l, W2r):
    src2d = edge_index[0].astype(jnp.int32).reshape(NW, E_PER_TILE)
    dst3d = edge_index[1].astype(jnp.int32).reshape(NW, CHUNKS, K)
    z128 = jnp.zeros((NP_, D), jnp.float32)
    ones = jnp.ones((K, D), jnp.float32)

    parts1, degp = _sc_segment_sum(x, src2d, dst3d, z128, ones, True)
    h = _tc_combine(parts1, degp, x, W1l, b1l, W1r, relu=True)
    (parts2,) = _sc_segment_sum(h, src2d, dst3d, z128, ones, False)
    out = _tc_combine(parts2, degp, h, W2l, b2l, W2r, relu=False)
    return out
